# baseline (speedup 1.0000x reference)
import functools

import jax
import jax.numpy as jnp
from jax import lax
from jax.experimental import pallas as pl
from jax.experimental.pallas import tpu as pltpu
from jax.experimental.pallas import tpu_sc as plsc

D_MODEL = 1024
D_HIDDEN = 4096
NUM_E = 8
NTOK = 8192

BB = 256
P_MAX = NTOK // BB + NUM_E
B_PAD = BB * P_MAX
HK = 1024
KC = D_HIDDEN // HK

SC_NC = 2
SC_NS = 16
SC_NW = SC_NC * SC_NS
SC_CH = 32
SC_JJ = NTOK // (SC_NW * SC_CH)


def _gate_body(x_ref, wg_ref, scores_ref, te_ref, ts_ref):
    xb = x_ref[...]
    logits = lax.dot_general(xb, wg_ref[...], (((1,), (1,)), ((), ())),
                             preferred_element_type=jnp.float32)
    m = jnp.max(logits, axis=1, keepdims=True)
    ex = jnp.exp(logits - m)
    s = ex / jnp.sum(ex, axis=1, keepdims=True)
    scores_ref[...] = s
    mx = jnp.max(s, axis=1, keepdims=True)
    ts_ref[...] = mx
    eids = lax.broadcasted_iota(jnp.int32, s.shape, 1)
    cand = jnp.where(s == mx, eids, NUM_E)
    te_ref[...] = jnp.min(cand, axis=1, keepdims=True)


def _gating(x, wg):
    blk = 1024
    return pl.pallas_call(
        _gate_body,
        grid=(NTOK // blk,),
        in_specs=[
            pl.BlockSpec((blk, D_MODEL), lambda i: (i, 0)),
            pl.BlockSpec((NUM_E, D_MODEL), lambda i: (0, 0)),
        ],
        out_specs=[
            pl.BlockSpec((blk, NUM_E), lambda i: (i, 0)),
            pl.BlockSpec((blk, 1), lambda i: (i, 0)),
            pl.BlockSpec((blk, 1), lambda i: (i, 0)),
        ],
        out_shape=[
            jax.ShapeDtypeStruct((NTOK, NUM_E), jnp.float32),
            jax.ShapeDtypeStruct((NTOK, 1), jnp.int32),
            jax.ShapeDtypeStruct((NTOK, 1), jnp.float32),
        ],
    )(x, wg)


def _route_body(te_ref, dst_ref, dens_ref, meta_ref):
    te = te_ref[...]
    r, c = te.shape
    ua = lax.broadcasted_iota(jnp.int32, (c, c), 0)
    ub = lax.broadcasted_iota(jnp.int32, (c, c), 1)
    umat = (ua <= ub).astype(jnp.float32)
    sa = lax.broadcasted_iota(jnp.int32, (r, r), 0)
    sb = lax.broadcasted_iota(jnp.int32, (r, r), 1)
    smat = (sb < sa).astype(jnp.float32)

    counts = []
    ranks = []
    for e in range(NUM_E):
        mask = (te == e)
        mf = mask.astype(jnp.float32)
        cc = lax.dot_general(mf, umat, (((1,), (0,)), ((), ())),
                             preferred_element_type=jnp.float32)
        rowtot = jnp.sum(mf, axis=1, keepdims=True)
        rowpfx = lax.dot_general(smat, rowtot, (((1,), (0,)), ((), ())),
                                 preferred_element_type=jnp.float32)
        rank0 = (rowpfx + cc).astype(jnp.int32) - 1
        ranks.append(rank0)
        counts.append(jnp.sum(mask.astype(jnp.int32)))

    starts = []
    acc = jnp.int32(0)
    for e in range(NUM_E):
        starts.append(acc)
        nb_e = (counts[e] + BB - 1) // BB
        acc = acc + nb_e
    ntot = acc

    dst = jnp.zeros((r, c), jnp.int32)
    for e in range(NUM_E):
        dst = jnp.where(te == e, BB * starts[e] + ranks[e], dst)
    dst_ref[...] = dst

    ri = lax.broadcasted_iota(jnp.int32, (NUM_E, NUM_E), 0)
    dens = jnp.zeros((NUM_E, NUM_E), jnp.float32)
    for e in range(NUM_E):
        dens = jnp.where(ri == e, counts[e].astype(jnp.float32) / NTOK, dens)
    dens_ref[...] = dens

    p_col = lax.broadcasted_iota(jnp.int32, (P_MAX, 1), 0)
    p_eff = jnp.minimum(p_col, ntot - 1)
    ci = lax.broadcasted_iota(jnp.int32, (P_MAX, NUM_E), 1)
    starts2d = jnp.zeros((P_MAX, NUM_E), jnp.int32)
    for e in range(NUM_E):
        starts2d = jnp.where(ci == e, starts[e], starts2d)
    ex_id = jnp.sum((p_eff >= starts2d).astype(jnp.int32), axis=1,
                    keepdims=True) - 1
    nb2d = jnp.zeros((P_MAX, NUM_E), jnp.int32)
    for e in range(NUM_E):
        nb_e = (counts[e] + BB - 1) // BB
        nb2d = jnp.where(ci == e, nb_e, nb2d)
    fe = jnp.max(((p_eff == starts2d) & (nb2d > 0)).astype(jnp.int32),
                 axis=1, keepdims=True)
    meta = jnp.where(ci == 0, p_eff, jnp.zeros((P_MAX, NUM_E), jnp.int32))
    meta = jnp.where(ci == 1, ex_id, meta)
    meta = jnp.where(ci == 2, fe, meta)
    meta_ref[...] = meta


def _routing(te64):
    return pl.pallas_call(
        _route_body,
        out_shape=[
            jax.ShapeDtypeStruct((NTOK // 128, 128), jnp.int32),
            jax.ShapeDtypeStruct((NUM_E, NUM_E), jnp.float32),
            jax.ShapeDtypeStruct((P_MAX, NUM_E), jnp.int32),
        ],
    )(te64)


def _sc_scatter_body(x_hbm, dst_hbm, xs_hbm, idx_v, rows_v, sem):
    wid = lax.axis_index("s") * SC_NC + lax.axis_index("c")
    base = wid * SC_CH * SC_JJ
    pltpu.sync_copy(dst_hbm.at[pl.ds(wid * SC_JJ, SC_JJ)], idx_v)
    for j in range(SC_JJ):
        pltpu.sync_copy(x_hbm.at[pl.ds(base + j * SC_CH, SC_CH)], rows_v)
        pltpu.async_copy(rows_v, xs_hbm.at[idx_v.at[j]], sem).wait()


def _sc_scatter(x, dst2):
    mesh = plsc.VectorSubcoreMesh(core_axis_name="c", subcore_axis_name="s",
                                  num_cores=SC_NC, num_subcores=SC_NS)
    fn = functools.partial(
        pl.kernel,
        out_type=jax.ShapeDtypeStruct((B_PAD, D_MODEL), jnp.float32),
        mesh=mesh,
        scratch_types=[
            pltpu.VMEM((SC_JJ, SC_CH), jnp.int32),
            pltpu.VMEM((SC_CH, D_MODEL), jnp.float32),
            pltpu.SemaphoreType.DMA,
        ],
    )(_sc_scatter_body)
    return fn(x, dst2)


def _sc_gather_body(ys_hbm, dst_hbm, out_hbm, idx_v, rows_v, sem):
    wid = lax.axis_index("s") * SC_NC + lax.axis_index("c")
    base = wid * SC_CH * SC_JJ
    pltpu.sync_copy(dst_hbm.at[pl.ds(wid * SC_JJ, SC_JJ)], idx_v)
    for j in range(SC_JJ):
        pltpu.async_copy(ys_hbm.at[idx_v.at[j]], rows_v, sem).wait()
        pltpu.sync_copy(rows_v, out_hbm.at[pl.ds(base + j * SC_CH, SC_CH)])


def _sc_gather(ys, dst2):
    mesh = plsc.VectorSubcoreMesh(core_axis_name="c", subcore_axis_name="s",
                                  num_cores=SC_NC, num_subcores=SC_NS)
    fn = functools.partial(
        pl.kernel,
        out_type=jax.ShapeDtypeStruct((NTOK, D_MODEL), jnp.float32),
        mesh=mesh,
        scratch_types=[
            pltpu.VMEM((SC_JJ, SC_CH), jnp.int32),
            pltpu.VMEM((SC_CH, D_MODEL), jnp.float32),
            pltpu.SemaphoreType.DMA,
        ],
    )(_sc_gather_body)
    return fn(ys, dst2)


def _ffn_body(prb_ref, pe_ref, fe_ref, xs_ref, w1_ref, b1_ref, w2_ref,
              b2_ref, wg_ref, ys_ref, w1b_ref, w2b_ref, acc_ref, ts_ref):
    p = pl.program_id(0)
    k = pl.program_id(1)

    @pl.when(fe_ref[p] == 1)
    def _():
        w1b_ref[pl.ds(k * HK, HK), :] = w1_ref[0].astype(jnp.bfloat16)
        w2b_ref[:, pl.ds(k * HK, HK)] = w2_ref[0].astype(jnp.bfloat16)

    xsb = xs_ref[...].astype(jnp.bfloat16)

    @pl.when(k == 0)
    def _():
        acc_ref[...] = jnp.zeros((BB, D_MODEL), jnp.float32)
        logits = lax.dot_general(xsb, wg_ref[...].astype(jnp.bfloat16),
                                 (((1,), (1,)), ((), ())),
                                 preferred_element_type=jnp.float32)
        m = jnp.max(logits, axis=1, keepdims=True)
        ts_ref[...] = 1.0 / jnp.sum(jnp.exp(logits - m), axis=1,
                                    keepdims=True)

    xw = lax.dot_general(xsb, w1b_ref[pl.ds(k * HK, HK), :],
                         (((1,), (1,)), ((), ())),
                         preferred_element_type=jnp.float32) + b1_ref[0]
    hk = jax.nn.gelu(xw, approximate=True).astype(jnp.bfloat16)
    acc_ref[...] += lax.dot_general(hk, w2b_ref[:, pl.ds(k * HK, HK)],
                                    (((1,), (1,)), ((), ())),
                                    preferred_element_type=jnp.float32)

    @pl.when(k == KC - 1)
    def _():
        ys_ref[...] = (acc_ref[...] + b2_ref[0]) * ts_ref[...]


def _grouped_ffn(prb, pe, fe, xs, w1, b1, w2, b2, wg):
    def kidx(k, fe_p):
        return fe_p * k + (1 - fe_p) * (KC - 1)

    return pl.pallas_call(
        _ffn_body,
        grid_spec=pltpu.PrefetchScalarGridSpec(
            num_scalar_prefetch=3,
            grid=(P_MAX, KC),
            in_specs=[
                pl.BlockSpec((BB, D_MODEL),
                             lambda p, k, prb, pe, fe: (prb[p], 0)),
                pl.BlockSpec((1, HK, D_MODEL),
                             lambda p, k, prb, pe, fe:
                             (pe[p], kidx(k, fe[p]), 0)),
                pl.BlockSpec((1, 1, HK),
                             lambda p, k, prb, pe, fe: (pe[p], 0, k)),
                pl.BlockSpec((1, D_MODEL, HK),
                             lambda p, k, prb, pe, fe:
                             (pe[p], 0, kidx(k, fe[p]))),
                pl.BlockSpec((1, 1, D_MODEL),
                             lambda p, k, prb, pe, fe: (pe[p], 0, 0)),
                pl.BlockSpec((NUM_E, D_MODEL),
                             lambda p, k, prb, pe, fe: (0, 0)),
            ],
            out_specs=pl.BlockSpec((BB, D_MODEL),
                                   lambda p, k, prb, pe, fe: (prb[p], 0)),
            scratch_shapes=[
                pltpu.VMEM((D_HIDDEN, D_MODEL), jnp.bfloat16),
                pltpu.VMEM((D_MODEL, D_HIDDEN), jnp.bfloat16),
                pltpu.VMEM((BB, D_MODEL), jnp.float32),
                pltpu.VMEM((BB, 1), jnp.float32),
            ],
        ),
        out_shape=jax.ShapeDtypeStruct((B_PAD, D_MODEL), jnp.float32),
    )(prb, pe, fe, xs, w1, b1.reshape(NUM_E, 1, D_HIDDEN), w2,
      b2.reshape(NUM_E, 1, D_MODEL), wg)


def _scale_body(raw_ref, ts_ref, out_ref):
    out_ref[...] = raw_ref[...] * ts_ref[...]


def _scale(raw, ts):
    blk = 1024
    return pl.pallas_call(
        _scale_body,
        grid=(NTOK // blk,),
        in_specs=[
            pl.BlockSpec((blk, D_MODEL), lambda i: (i, 0)),
            pl.BlockSpec((blk, 1), lambda i: (i, 0)),
        ],
        out_specs=pl.BlockSpec((blk, D_MODEL), lambda i: (i, 0)),
        out_shape=jax.ShapeDtypeStruct((NTOK, D_MODEL), jnp.float32),
    )(raw, ts)


def kernel(x, Wg, W1, b1, W2, b2):
    scores, te, ts = _gating(x, Wg)
    te64 = te.reshape(NTOK // 128, 128)
    dst64, dens, meta = _routing(te64)
    dst2 = dst64.reshape(SC_NW * SC_JJ, SC_CH)
    prb = meta[:, 0]
    pe = meta[:, 1]
    fe = meta[:, 2]
    xs = _sc_scatter(x, dst2)
    ys = _grouped_ffn(prb, pe, fe, xs, W1, b1, W2, b2, Wg)
    out = _sc_gather(ys, dst2)
    return (out, scores, te.reshape(NTOK), dens[:, 0])


# speedup vs baseline: 1.0626x; 1.0626x over previous
import functools

import jax
import jax.numpy as jnp
from jax import lax
from jax.experimental import pallas as pl
from jax.experimental.pallas import tpu as pltpu
from jax.experimental.pallas import tpu_sc as plsc

D_MODEL = 1024
D_HIDDEN = 4096
NUM_E = 8
NTOK = 8192

BB = 256
P_MAX = NTOK // BB + NUM_E
B_PAD = BB * P_MAX
HK = 1024
KC = D_HIDDEN // HK

SC_NC = 2
SC_NS = 16
SC_NW = SC_NC * SC_NS
SC_CH = 32
SC_JJ = NTOK // (SC_NW * SC_CH)


def _gate_body(x_ref, wg_ref, scores_ref, te_ref, ts_ref):
    xb = x_ref[...]
    logits = lax.dot_general(xb, wg_ref[...], (((1,), (1,)), ((), ())),
                             preferred_element_type=jnp.float32)
    m = jnp.max(logits, axis=1, keepdims=True)
    ex = jnp.exp(logits - m)
    s = ex / jnp.sum(ex, axis=1, keepdims=True)
    scores_ref[...] = s
    mx = jnp.max(s, axis=1, keepdims=True)
    ts_ref[...] = mx
    eids = lax.broadcasted_iota(jnp.int32, s.shape, 1)
    cand = jnp.where(s == mx, eids, NUM_E)
    te_ref[...] = jnp.min(cand, axis=1, keepdims=True)


def _gating(x, wg):
    blk = 1024
    return pl.pallas_call(
        _gate_body,
        grid=(NTOK // blk,),
        in_specs=[
            pl.BlockSpec((blk, D_MODEL), lambda i: (i, 0)),
            pl.BlockSpec((NUM_E, D_MODEL), lambda i: (0, 0)),
        ],
        out_specs=[
            pl.BlockSpec((blk, NUM_E), lambda i: (i, 0)),
            pl.BlockSpec((blk, 1), lambda i: (i, 0)),
            pl.BlockSpec((blk, 1), lambda i: (i, 0)),
        ],
        out_shape=[
            jax.ShapeDtypeStruct((NTOK, NUM_E), jnp.float32),
            jax.ShapeDtypeStruct((NTOK, 1), jnp.int32),
            jax.ShapeDtypeStruct((NTOK, 1), jnp.float32),
        ],
    )(x, wg)


def _route_body(te_ref, dst_ref, dens_ref, meta_ref):
    te = te_ref[...]
    r, c = te.shape
    ua = lax.broadcasted_iota(jnp.int32, (c, c), 0)
    ub = lax.broadcasted_iota(jnp.int32, (c, c), 1)
    umat = (ua <= ub).astype(jnp.float32)
    sa = lax.broadcasted_iota(jnp.int32, (r, r), 0)
    sb = lax.broadcasted_iota(jnp.int32, (r, r), 1)
    smat = (sb < sa).astype(jnp.float32)

    counts = []
    ranks = []
    for e in range(NUM_E):
        mask = (te == e)
        mf = mask.astype(jnp.float32)
        cc = lax.dot_general(mf, umat, (((1,), (0,)), ((), ())),
                             preferred_element_type=jnp.float32)
        rowtot = jnp.sum(mf, axis=1, keepdims=True)
        rowpfx = lax.dot_general(smat, rowtot, (((1,), (0,)), ((), ())),
                                 preferred_element_type=jnp.float32)
        rank0 = (rowpfx + cc).astype(jnp.int32) - 1
        ranks.append(rank0)
        counts.append(jnp.sum(mask.astype(jnp.int32)))

    starts = []
    acc = jnp.int32(0)
    for e in range(NUM_E):
        starts.append(acc)
        nb_e = (counts[e] + BB - 1) // BB
        acc = acc + nb_e
    ntot = acc

    dst = jnp.zeros((r, c), jnp.int32)
    for e in range(NUM_E):
        dst = jnp.where(te == e, BB * starts[e] + ranks[e], dst)
    dst_ref[...] = dst

    ri = lax.broadcasted_iota(jnp.int32, (NUM_E, NUM_E), 0)
    dens = jnp.zeros((NUM_E, NUM_E), jnp.float32)
    for e in range(NUM_E):
        dens = jnp.where(ri == e, counts[e].astype(jnp.float32) / NTOK, dens)
    dens_ref[...] = dens

    p_col = lax.broadcasted_iota(jnp.int32, (P_MAX, 1), 0)
    p_eff = jnp.minimum(p_col, ntot - 1)
    ci = lax.broadcasted_iota(jnp.int32, (P_MAX, NUM_E), 1)
    starts2d = jnp.zeros((P_MAX, NUM_E), jnp.int32)
    for e in range(NUM_E):
        starts2d = jnp.where(ci == e, starts[e], starts2d)
    ex_id = jnp.sum((p_eff >= starts2d).astype(jnp.int32), axis=1,
                    keepdims=True) - 1
    nb2d = jnp.zeros((P_MAX, NUM_E), jnp.int32)
    for e in range(NUM_E):
        nb_e = (counts[e] + BB - 1) // BB
        nb2d = jnp.where(ci == e, nb_e, nb2d)
    fe = jnp.max(((p_eff == starts2d) & (nb2d > 0)).astype(jnp.int32),
                 axis=1, keepdims=True)
    meta = jnp.where(ci == 0, p_eff, jnp.zeros((P_MAX, NUM_E), jnp.int32))
    meta = jnp.where(ci == 1, ex_id, meta)
    meta = jnp.where(ci == 2, fe, meta)
    meta_ref[...] = meta


def _routing(te64):
    return pl.pallas_call(
        _route_body,
        out_shape=[
            jax.ShapeDtypeStruct((NTOK // 128, 128), jnp.int32),
            jax.ShapeDtypeStruct((NUM_E, NUM_E), jnp.float32),
            jax.ShapeDtypeStruct((P_MAX, NUM_E), jnp.int32),
        ],
    )(te64)


def _sc_scatter_body(x_hbm, dst_hbm, xs_hbm, idx_v, rows_v, sem_in, sem_out):
    wid = lax.axis_index("s") * SC_NC + lax.axis_index("c")
    base = wid * SC_CH * SC_JJ
    pltpu.sync_copy(dst_hbm.at[pl.ds(wid * SC_JJ, SC_JJ)], idx_v)
    pltpu.sync_copy(x_hbm.at[pl.ds(base, SC_CH)], rows_v.at[0])
    for j in range(SC_JJ):
        if j < SC_JJ - 1:
            cin = pltpu.async_copy(
                x_hbm.at[pl.ds(base + (j + 1) * SC_CH, SC_CH)],
                rows_v.at[(j + 1) % 2], sem_in)
        cout = pltpu.async_copy(rows_v.at[j % 2], xs_hbm.at[idx_v.at[j]],
                                sem_out)
        if j < SC_JJ - 1:
            cin.wait()
        cout.wait()


def _sc_scatter(x, dst2):
    mesh = plsc.VectorSubcoreMesh(core_axis_name="c", subcore_axis_name="s",
                                  num_cores=SC_NC, num_subcores=SC_NS)
    fn = functools.partial(
        pl.kernel,
        out_type=jax.ShapeDtypeStruct((B_PAD, D_MODEL), jnp.float32),
        mesh=mesh,
        scratch_types=[
            pltpu.VMEM((SC_JJ, SC_CH), jnp.int32),
            pltpu.VMEM((2, SC_CH, D_MODEL), jnp.float32),
            pltpu.SemaphoreType.DMA,
            pltpu.SemaphoreType.DMA,
        ],
    )(_sc_scatter_body)
    return fn(x, dst2)


def _sc_gather_body(ys_hbm, dst_hbm, out_hbm, idx_v, rows_v, sem_in, sem_out):
    wid = lax.axis_index("s") * SC_NC + lax.axis_index("c")
    base = wid * SC_CH * SC_JJ
    pltpu.sync_copy(dst_hbm.at[pl.ds(wid * SC_JJ, SC_JJ)], idx_v)
    pltpu.async_copy(ys_hbm.at[idx_v.at[0]], rows_v.at[0], sem_in).wait()
    for j in range(SC_JJ):
        if j < SC_JJ - 1:
            cin = pltpu.async_copy(ys_hbm.at[idx_v.at[j + 1]],
                                   rows_v.at[(j + 1) % 2], sem_in)
        cout = pltpu.async_copy(
            rows_v.at[j % 2], out_hbm.at[pl.ds(base + j * SC_CH, SC_CH)],
            sem_out)
        if j < SC_JJ - 1:
            cin.wait()
        cout.wait()


def _sc_gather(ys, dst2):
    mesh = plsc.VectorSubcoreMesh(core_axis_name="c", subcore_axis_name="s",
                                  num_cores=SC_NC, num_subcores=SC_NS)
    fn = functools.partial(
        pl.kernel,
        out_type=jax.ShapeDtypeStruct((NTOK, D_MODEL), jnp.float32),
        mesh=mesh,
        scratch_types=[
            pltpu.VMEM((SC_JJ, SC_CH), jnp.int32),
            pltpu.VMEM((2, SC_CH, D_MODEL), jnp.float32),
            pltpu.SemaphoreType.DMA,
            pltpu.SemaphoreType.DMA,
        ],
    )(_sc_gather_body)
    return fn(ys, dst2)


def _fc1_body(prb_ref, pe_ref, fe_ref, xs_ref, w1_ref, b1_ref, wg_ref,
              h_ref, ts_ref, w1b_ref):
    p = pl.program_id(0)

    @pl.when(fe_ref[p] == 1)
    def _():
        w1b_ref[...] = w1_ref[0].astype(jnp.bfloat16)

    xsb = xs_ref[...].astype(jnp.bfloat16)
    xw = lax.dot_general(xsb, w1b_ref[...], (((1,), (1,)), ((), ())),
                         preferred_element_type=jnp.float32) + b1_ref[0]
    h_ref[...] = jax.nn.gelu(xw, approximate=True).astype(jnp.bfloat16)
    logits = lax.dot_general(xsb, wg_ref[...].astype(jnp.bfloat16),
                             (((1,), (1,)), ((), ())),
                             preferred_element_type=jnp.float32)
    m = jnp.max(logits, axis=1, keepdims=True)
    ts_ref[...] = 1.0 / jnp.sum(jnp.exp(logits - m), axis=1, keepdims=True)


def _fc2_body(prb_ref, pe_ref, fe_ref, h_ref, w2_ref, b2_ref, ts_ref,
              ys_ref, w2b_ref):
    p = pl.program_id(0)

    @pl.when(fe_ref[p] == 1)
    def _():
        w2b_ref[...] = w2_ref[0].astype(jnp.bfloat16)

    y = lax.dot_general(h_ref[...], w2b_ref[...], (((1,), (1,)), ((), ())),
                        preferred_element_type=jnp.float32) + b2_ref[0]
    ys_ref[...] = y * ts_ref[...]


def _grouped_ffn(prb, pe, fe, xs, w1, b1, w2, b2, wg):
    h, ts_s = pl.pallas_call(
        _fc1_body,
        grid_spec=pltpu.PrefetchScalarGridSpec(
            num_scalar_prefetch=3,
            grid=(P_MAX,),
            in_specs=[
                pl.BlockSpec((BB, D_MODEL), lambda p, prb, pe, fe: (prb[p], 0)),
                pl.BlockSpec((1, D_HIDDEN, D_MODEL),
                             lambda p, prb, pe, fe: (pe[p], 0, 0)),
                pl.BlockSpec((1, 1, D_HIDDEN),
                             lambda p, prb, pe, fe: (pe[p], 0, 0)),
                pl.BlockSpec((NUM_E, D_MODEL), lambda p, prb, pe, fe: (0, 0)),
            ],
            out_specs=[
                pl.BlockSpec((BB, D_HIDDEN), lambda p, prb, pe, fe: (prb[p], 0)),
                pl.BlockSpec((BB, 1), lambda p, prb, pe, fe: (prb[p], 0)),
            ],
            scratch_shapes=[pltpu.VMEM((D_HIDDEN, D_MODEL), jnp.bfloat16)],
        ),
        out_shape=[
            jax.ShapeDtypeStruct((B_PAD, D_HIDDEN), jnp.bfloat16),
            jax.ShapeDtypeStruct((B_PAD, 1), jnp.float32),
        ],
    )(prb, pe, fe, xs, w1, b1.reshape(NUM_E, 1, D_HIDDEN), wg)
    return pl.pallas_call(
        _fc2_body,
        grid_spec=pltpu.PrefetchScalarGridSpec(
            num_scalar_prefetch=3,
            grid=(P_MAX,),
            in_specs=[
                pl.BlockSpec((BB, D_HIDDEN), lambda p, prb, pe, fe: (prb[p], 0)),
                pl.BlockSpec((1, D_MODEL, D_HIDDEN),
                             lambda p, prb, pe, fe: (pe[p], 0, 0)),
                pl.BlockSpec((1, 1, D_MODEL),
                             lambda p, prb, pe, fe: (pe[p], 0, 0)),
                pl.BlockSpec((BB, 1), lambda p, prb, pe, fe: (prb[p], 0)),
            ],
            out_specs=pl.BlockSpec((BB, D_MODEL),
                                   lambda p, prb, pe, fe: (prb[p], 0)),
            scratch_shapes=[pltpu.VMEM((D_MODEL, D_HIDDEN), jnp.bfloat16)],
        ),
        out_shape=jax.ShapeDtypeStruct((B_PAD, D_MODEL), jnp.float32),
    )(prb, pe, fe, h, w2, b2.reshape(NUM_E, 1, D_MODEL), ts_s)


def _scale_body(raw_ref, ts_ref, out_ref):
    out_ref[...] = raw_ref[...] * ts_ref[...]


def _scale(raw, ts):
    blk = 1024
    return pl.pallas_call(
        _scale_body,
        grid=(NTOK // blk,),
        in_specs=[
            pl.BlockSpec((blk, D_MODEL), lambda i: (i, 0)),
            pl.BlockSpec((blk, 1), lambda i: (i, 0)),
        ],
        out_specs=pl.BlockSpec((blk, D_MODEL), lambda i: (i, 0)),
        out_shape=jax.ShapeDtypeStruct((NTOK, D_MODEL), jnp.float32),
    )(raw, ts)


def kernel(x, Wg, W1, b1, W2, b2):
    scores, te, ts = _gating(x, Wg)
    te64 = te.reshape(NTOK // 128, 128)
    dst64, dens, meta = _routing(te64)
    dst2 = dst64.reshape(SC_NW * SC_JJ, SC_CH)
    prb = meta[:, 0]
    pe = meta[:, 1]
    fe = meta[:, 2]
    xs = _sc_scatter(x, dst2)
    ys = _grouped_ffn(prb, pe, fe, xs, W1, b1, W2, b2, Wg)
    out = _sc_gather(ys, dst2)
    return (out, scores, te.reshape(NTOK), dens[:, 0])


# speedup vs baseline: 1.0628x; 1.0001x over previous
import functools

import jax
import jax.numpy as jnp
from jax import lax
from jax.experimental import pallas as pl
from jax.experimental.pallas import tpu as pltpu
from jax.experimental.pallas import tpu_sc as plsc

D_MODEL = 1024
D_HIDDEN = 4096
NUM_E = 8
NTOK = 8192

BB = 256
P_MAX = NTOK // BB + NUM_E
B_PAD = BB * P_MAX
HK = 1024
KC = D_HIDDEN // HK

SC_NC = 2
SC_NS = 16
SC_NW = SC_NC * SC_NS
SC_CH = 32
SC_JJ = NTOK // (SC_NW * SC_CH)


def _gate_body(x_ref, wg_ref, scores_ref, te_ref):
    xb = x_ref[...]
    logits = lax.dot_general(xb, wg_ref[...], (((1,), (1,)), ((), ())),
                             preferred_element_type=jnp.float32)
    m = jnp.max(logits, axis=1, keepdims=True)
    ex = jnp.exp(logits - m)
    s = ex / jnp.sum(ex, axis=1, keepdims=True)
    scores_ref[...] = s
    mx = jnp.max(s, axis=1, keepdims=True)
    eids = lax.broadcasted_iota(jnp.int32, s.shape, 1)
    cand = jnp.where(s == mx, eids, NUM_E)
    te_ref[...] = jnp.min(cand, axis=1, keepdims=True)


def _gating(x, wg):
    blk = 1024
    return pl.pallas_call(
        _gate_body,
        grid=(NTOK // blk,),
        in_specs=[
            pl.BlockSpec((blk, D_MODEL), lambda i: (i, 0)),
            pl.BlockSpec((NUM_E, D_MODEL), lambda i: (0, 0)),
        ],
        out_specs=[
            pl.BlockSpec((blk, NUM_E), lambda i: (i, 0)),
            pl.BlockSpec((blk, 1), lambda i: (i, 0)),
        ],
        out_shape=[
            jax.ShapeDtypeStruct((NTOK, NUM_E), jnp.float32),
            jax.ShapeDtypeStruct((NTOK, 1), jnp.int32),
        ],
    )(x, wg)


def _route_body(te_ref, dst_ref, dens_ref, meta_ref):
    te = te_ref[...]
    r, c = te.shape
    ua = lax.broadcasted_iota(jnp.int32, (c, c), 0)
    ub = lax.broadcasted_iota(jnp.int32, (c, c), 1)
    umat = (ua <= ub).astype(jnp.float32)
    sa = lax.broadcasted_iota(jnp.int32, (r, r), 0)
    sb = lax.broadcasted_iota(jnp.int32, (r, r), 1)
    smat = (sb < sa).astype(jnp.float32)

    counts = []
    ranks = []
    for e in range(NUM_E):
        mask = (te == e)
        mf = mask.astype(jnp.float32)
        cc = lax.dot_general(mf, umat, (((1,), (0,)), ((), ())),
                             preferred_element_type=jnp.float32)
        rowtot = jnp.sum(mf, axis=1, keepdims=True)
        rowpfx = lax.dot_general(smat, rowtot, (((1,), (0,)), ((), ())),
                                 preferred_element_type=jnp.float32)
        rank0 = (rowpfx + cc).astype(jnp.int32) - 1
        ranks.append(rank0)
        counts.append(jnp.sum(mask.astype(jnp.int32)))

    starts = []
    acc = jnp.int32(0)
    for e in range(NUM_E):
        starts.append(acc)
        nb_e = (counts[e] + BB - 1) // BB
        acc = acc + nb_e
    ntot = acc

    dst = jnp.zeros((r, c), jnp.int32)
    for e in range(NUM_E):
        dst = jnp.where(te == e, BB * starts[e] + ranks[e], dst)
    dst_ref[...] = dst

    ri = lax.broadcasted_iota(jnp.int32, (NUM_E, NUM_E), 0)
    dens = jnp.zeros((NUM_E, NUM_E), jnp.float32)
    for e in range(NUM_E):
        dens = jnp.where(ri == e, counts[e].astype(jnp.float32) / NTOK, dens)
    dens_ref[...] = dens

    p_col = lax.broadcasted_iota(jnp.int32, (P_MAX, 1), 0)
    p_eff = jnp.minimum(p_col, ntot - 1)
    ci = lax.broadcasted_iota(jnp.int32, (P_MAX, NUM_E), 1)
    starts2d = jnp.zeros((P_MAX, NUM_E), jnp.int32)
    for e in range(NUM_E):
        starts2d = jnp.where(ci == e, starts[e], starts2d)
    ex_id = jnp.sum((p_eff >= starts2d).astype(jnp.int32), axis=1,
                    keepdims=True) - 1
    nb2d = jnp.zeros((P_MAX, NUM_E), jnp.int32)
    for e in range(NUM_E):
        nb_e = (counts[e] + BB - 1) // BB
        nb2d = jnp.where(ci == e, nb_e, nb2d)
    fe = jnp.max(((p_eff == starts2d) & (nb2d > 0)).astype(jnp.int32),
                 axis=1, keepdims=True)
    meta = jnp.where(ci == 0, p_eff, jnp.zeros((P_MAX, NUM_E), jnp.int32))
    meta = jnp.where(ci == 1, ex_id, meta)
    meta = jnp.where(ci == 2, fe, meta)
    meta_ref[...] = meta


def _routing(te64):
    return pl.pallas_call(
        _route_body,
        out_shape=[
            jax.ShapeDtypeStruct((NTOK // 128, 128), jnp.int32),
            jax.ShapeDtypeStruct((NUM_E, NUM_E), jnp.float32),
            jax.ShapeDtypeStruct((P_MAX, NUM_E), jnp.int32),
        ],
    )(te64)


def _sc_scatter_body(x_hbm, dst_hbm, xs_hbm, idx_v, rows_v, sem_in, sem_out):
    wid = lax.axis_index("s") * SC_NC + lax.axis_index("c")
    base = wid * SC_CH * SC_JJ
    pltpu.sync_copy(dst_hbm.at[pl.ds(wid * SC_JJ, SC_JJ)], idx_v)
    pltpu.sync_copy(x_hbm.at[pl.ds(base, SC_CH)], rows_v.at[0])
    for j in range(SC_JJ):
        if j < SC_JJ - 1:
            cin = pltpu.async_copy(
                x_hbm.at[pl.ds(base + (j + 1) * SC_CH, SC_CH)],
                rows_v.at[(j + 1) % 2], sem_in)
        cout = pltpu.async_copy(rows_v.at[j % 2], xs_hbm.at[idx_v.at[j]],
                                sem_out)
        if j < SC_JJ - 1:
            cin.wait()
        cout.wait()


def _sc_scatter(x, dst2):
    mesh = plsc.VectorSubcoreMesh(core_axis_name="c", subcore_axis_name="s",
                                  num_cores=SC_NC, num_subcores=SC_NS)
    fn = functools.partial(
        pl.kernel,
        out_type=jax.ShapeDtypeStruct((B_PAD, D_MODEL), jnp.float32),
        mesh=mesh,
        scratch_types=[
            pltpu.VMEM((SC_JJ, SC_CH), jnp.int32),
            pltpu.VMEM((2, SC_CH, D_MODEL), jnp.float32),
            pltpu.SemaphoreType.DMA,
            pltpu.SemaphoreType.DMA,
        ],
    )(_sc_scatter_body)
    return fn(x, dst2)


def _sc_gather_body(ys_hbm, dst_hbm, out_hbm, idx_v, rows_v, sem_in, sem_out):
    wid = lax.axis_index("s") * SC_NC + lax.axis_index("c")
    base = wid * SC_CH * SC_JJ
    pltpu.sync_copy(dst_hbm.at[pl.ds(wid * SC_JJ, SC_JJ)], idx_v)
    pltpu.async_copy(ys_hbm.at[idx_v.at[0]], rows_v.at[0], sem_in).wait()
    for j in range(SC_JJ):
        if j < SC_JJ - 1:
            cin = pltpu.async_copy(ys_hbm.at[idx_v.at[j + 1]],
                                   rows_v.at[(j + 1) % 2], sem_in)
        cout = pltpu.async_copy(
            rows_v.at[j % 2], out_hbm.at[pl.ds(base + j * SC_CH, SC_CH)],
            sem_out)
        if j < SC_JJ - 1:
            cin.wait()
        cout.wait()


def _sc_gather(ys, dst2):
    mesh = plsc.VectorSubcoreMesh(core_axis_name="c", subcore_axis_name="s",
                                  num_cores=SC_NC, num_subcores=SC_NS)
    fn = functools.partial(
        pl.kernel,
        out_type=jax.ShapeDtypeStruct((NTOK, D_MODEL), jnp.float32),
        mesh=mesh,
        scratch_types=[
            pltpu.VMEM((SC_JJ, SC_CH), jnp.int32),
            pltpu.VMEM((2, SC_CH, D_MODEL), jnp.float32),
            pltpu.SemaphoreType.DMA,
            pltpu.SemaphoreType.DMA,
        ],
    )(_sc_gather_body)
    return fn(ys, dst2)


def _fc1_body(prb_ref, pe_ref, fe_ref, xs_ref, w1_ref, b1_ref, wg_ref,
              h_ref, ts_ref, w1b_ref):
    p = pl.program_id(0)

    @pl.when(fe_ref[p] == 1)
    def _():
        w1b_ref[...] = w1_ref[0].astype(jnp.bfloat16)

    xsb = xs_ref[...].astype(jnp.bfloat16)
    xw = lax.dot_general(xsb, w1b_ref[...], (((1,), (1,)), ((), ())),
                         preferred_element_type=jnp.float32) + b1_ref[0]
    h_ref[...] = jax.nn.gelu(xw, approximate=True).astype(jnp.bfloat16)
    logits = lax.dot_general(xsb, wg_ref[...].astype(jnp.bfloat16),
                             (((1,), (1,)), ((), ())),
                             preferred_element_type=jnp.float32)
    m = jnp.max(logits, axis=1, keepdims=True)
    ts_ref[...] = 1.0 / jnp.sum(jnp.exp(logits - m), axis=1, keepdims=True)


def _fc2_body(prb_ref, pe_ref, fe_ref, h_ref, w2_ref, b2_ref, ts_ref,
              ys_ref, w2b_ref):
    p = pl.program_id(0)

    @pl.when(fe_ref[p] == 1)
    def _():
        w2b_ref[...] = w2_ref[0].astype(jnp.bfloat16)

    y = lax.dot_general(h_ref[...], w2b_ref[...], (((1,), (1,)), ((), ())),
                        preferred_element_type=jnp.float32) + b2_ref[0]
    ys_ref[...] = y * ts_ref[...]


def _grouped_ffn(prb, pe, fe, xs, w1, b1, w2, b2, wg):
    h, ts_s = pl.pallas_call(
        _fc1_body,
        grid_spec=pltpu.PrefetchScalarGridSpec(
            num_scalar_prefetch=3,
            grid=(P_MAX,),
            in_specs=[
                pl.BlockSpec((BB, D_MODEL), lambda p, prb, pe, fe: (prb[p], 0)),
                pl.BlockSpec((1, D_HIDDEN, D_MODEL),
                             lambda p, prb, pe, fe: (pe[p], 0, 0)),
                pl.BlockSpec((1, 1, D_HIDDEN),
                             lambda p, prb, pe, fe: (pe[p], 0, 0)),
                pl.BlockSpec((NUM_E, D_MODEL), lambda p, prb, pe, fe: (0, 0)),
            ],
            out_specs=[
                pl.BlockSpec((BB, D_HIDDEN), lambda p, prb, pe, fe: (prb[p], 0)),
                pl.BlockSpec((BB, 1), lambda p, prb, pe, fe: (prb[p], 0)),
            ],
            scratch_shapes=[pltpu.VMEM((D_HIDDEN, D_MODEL), jnp.bfloat16)],
        ),
        out_shape=[
            jax.ShapeDtypeStruct((B_PAD, D_HIDDEN), jnp.bfloat16),
            jax.ShapeDtypeStruct((B_PAD, 1), jnp.float32),
        ],
    )(prb, pe, fe, xs, w1, b1.reshape(NUM_E, 1, D_HIDDEN), wg)
    return pl.pallas_call(
        _fc2_body,
        grid_spec=pltpu.PrefetchScalarGridSpec(
            num_scalar_prefetch=3,
            grid=(P_MAX,),
            in_specs=[
                pl.BlockSpec((BB, D_HIDDEN), lambda p, prb, pe, fe: (prb[p], 0)),
                pl.BlockSpec((1, D_MODEL, D_HIDDEN),
                             lambda p, prb, pe, fe: (pe[p], 0, 0)),
                pl.BlockSpec((1, 1, D_MODEL),
                             lambda p, prb, pe, fe: (pe[p], 0, 0)),
                pl.BlockSpec((BB, 1), lambda p, prb, pe, fe: (prb[p], 0)),
            ],
            out_specs=pl.BlockSpec((BB, D_MODEL),
                                   lambda p, prb, pe, fe: (prb[p], 0)),
            scratch_shapes=[pltpu.VMEM((D_MODEL, D_HIDDEN), jnp.bfloat16)],
        ),
        out_shape=jax.ShapeDtypeStruct((B_PAD, D_MODEL), jnp.float32),
    )(prb, pe, fe, h, w2, b2.reshape(NUM_E, 1, D_MODEL), ts_s)


def kernel(x, Wg, W1, b1, W2, b2):
    scores, te = _gating(x, Wg)
    te64 = te.reshape(NTOK // 128, 128)
    dst64, dens, meta = _routing(te64)
    dst2 = dst64.reshape(SC_NW * SC_JJ, SC_CH)
    prb = meta[:, 0]
    pe = meta[:, 1]
    fe = meta[:, 2]
    xs = _sc_scatter(x, dst2)
    ys = _grouped_ffn(prb, pe, fe, xs, W1, b1, W2, b2, Wg)
    out = _sc_gather(ys, dst2)
    return (out, scores, te.reshape(NTOK), dens[:, 0])
